# Initial kernel scaffold; baseline (speedup 1.0000x reference)
import jax
import jax.numpy as jnp
from jax.experimental import pallas as pl


def kernel(x, edge_index, ids, W_in, Wg, a_src, a_dst, Wi, Uh, bi, bh, Wfc, bfc, Wsc, bsc):
    raise NotImplementedError("write your pallas kernel here")


# speedup vs baseline: 1.0307x; 1.0307x over previous
import jax
import jax.numpy as jnp
from jax.experimental import pallas as pl

_N = 10000
_E = 320000
_NFEAT = 128
_NHID = 256
_H = 8
_DH = _NHID // _H
_T = 8
_ALPHA = 0.2

_ROWB = 2000
_EDGEB = 3200


def _mm_kernel(x_ref, w_ref, o_ref):
    o_ref[...] = jnp.dot(x_ref[...], w_ref[...], preferred_element_type=jnp.float32)


def _dense1_kernel(feat_ref, hp_ref, wg_ref, as_ref, ad_ref, whh_ref, es_ref, ed_ref):
    inp = feat_ref[...] + hp_ref[...]
    whh = jnp.dot(inp, wg_ref[...], preferred_element_type=jnp.float32)
    whh_ref[...] = whh
    es_ref[...] = jnp.dot(whh, as_ref[...], preferred_element_type=jnp.float32)
    ed_ref[...] = jnp.dot(whh, ad_ref[...], preferred_element_type=jnp.float32)


def _leaky_kernel(g_ref, o_ref):
    g = g_ref[...]
    o_ref[...] = jnp.where(g >= 0.0, g, _ALPHA * g)


def _ee_kernel(e_ref, m_ref, o_ref):
    o_ref[...] = jnp.exp(e_ref[...] - m_ref[...])


def _att_kernel(ee_ref, s_ref, o_ref):
    o_ref[...] = ee_ref[...] / (s_ref[...] + 1e-9)


def _dense2_kernel(g_ref, hp_ref, wi_ref, uh_ref, bi_ref, bh_ref, m_ref, o_ref):
    g = g_ref[...]
    act = jnp.where(g > 0.0, g, jnp.exp(g) - 1.0)
    hp = hp_ref[...]
    gi = jnp.dot(act, wi_ref[...], preferred_element_type=jnp.float32) + bi_ref[...]
    gh = jnp.dot(hp, uh_ref[...], preferred_element_type=jnp.float32) + bh_ref[...]
    r = jax.nn.sigmoid(gi[:, 0:_NHID] + gh[:, 0:_NHID])
    z = jax.nn.sigmoid(gi[:, _NHID:2 * _NHID] + gh[:, _NHID:2 * _NHID])
    nn_ = jnp.tanh(gi[:, 2 * _NHID:] + r * gh[:, 2 * _NHID:])
    hnew = (1.0 - z) * nn_ + z * hp
    o_ref[...] = jnp.where(m_ref[...] > 0.5, hnew, hp)


def _readout_kernel(hp_ref, ids_ref, wfc_ref, bfc_ref, wsc_ref, bsc_ref,
                    s1_ref, s2_ref):
    ids = ids_ref[...]
    tt = jax.lax.broadcasted_iota(jnp.int32, (1, _T), 1)
    oh = (ids == tt).astype(jnp.float32)
    hp = hp_ref[...]
    sums = jax.lax.dot_general(oh, hp, (((0,), (0,)), ((), ())),
                               preferred_element_type=jnp.float32)
    ones = jnp.ones((_N, 1), jnp.float32)
    cnts = jax.lax.dot_general(oh, ones, (((0,), (0,)), ((), ())),
                               preferred_element_type=jnp.float32)
    means = sums / jnp.maximum(cnts, 1.0)
    h1 = means[0:1] + means[2:3] + means[4:5] + means[6:7]
    h2 = means[1:2] + means[3:4] + means[5:6] + means[7:8]
    a1 = jnp.maximum(jnp.dot(h1, wfc_ref[...], preferred_element_type=jnp.float32)
                     + bfc_ref[...], 0.0)
    a2 = jnp.maximum(jnp.dot(h2, wfc_ref[...], preferred_element_type=jnp.float32)
                     + bfc_ref[...], 0.0)
    s1_ref[...] = jnp.dot(a1, wsc_ref[...], preferred_element_type=jnp.float32) + bsc_ref[...]
    s2_ref[...] = jnp.dot(a2, wsc_ref[...], preferred_element_type=jnp.float32) + bsc_ref[...]


def _row_spec(cols):
    return pl.BlockSpec((_ROWB, cols), lambda i: (i, 0))


def _full_spec(rows, cols):
    return pl.BlockSpec((rows, cols), lambda i: (0, 0))


def _edge_spec(cols):
    return pl.BlockSpec((_EDGEB, cols), lambda i: (i, 0))


def _edge_ew(fn, out_like, *args):
    n_in = len(args)
    return pl.pallas_call(
        fn,
        grid=(_E // _EDGEB,),
        in_specs=[_edge_spec(_H)] * n_in,
        out_specs=_edge_spec(_H),
        out_shape=jax.ShapeDtypeStruct((_E, _H), jnp.float32),
    )(*args)


@jax.jit
def kernel(x, edge_index, ids, W_in, Wg, a_src, a_dst, Wi, Uh, bi, bh, Wfc, bfc, Wsc, bsc):
    src, dst = edge_index[0], edge_index[1]

    eye = jnp.eye(_H, dtype=jnp.float32)
    A_src = (a_src[:, :, None] * eye[:, None, :]).reshape(_NHID, _H)
    A_dst = (a_dst[:, :, None] * eye[:, None, :]).reshape(_NHID, _H)
    bi2 = bi.reshape(1, 3 * _NHID)
    bh2 = bh.reshape(1, 3 * _NHID)
    ids2 = ids.reshape(_N, 1)

    feat = pl.pallas_call(
        _mm_kernel,
        grid=(_N // _ROWB,),
        in_specs=[_row_spec(_NFEAT), _full_spec(_NFEAT, _NHID)],
        out_specs=_row_spec(_NHID),
        out_shape=jax.ShapeDtypeStruct((_N, _NHID), jnp.float32),
    )(x, W_in)

    hp = jnp.zeros((_N, _NHID), jnp.float32)

    dense1 = pl.pallas_call(
        _dense1_kernel,
        grid=(_N // _ROWB,),
        in_specs=[_row_spec(_NHID), _row_spec(_NHID), _full_spec(_NHID, _NHID),
                  _full_spec(_NHID, _H), _full_spec(_NHID, _H)],
        out_specs=[_row_spec(_NHID), _row_spec(_H), _row_spec(_H)],
        out_shape=[jax.ShapeDtypeStruct((_N, _NHID), jnp.float32),
                   jax.ShapeDtypeStruct((_N, _H), jnp.float32),
                   jax.ShapeDtypeStruct((_N, _H), jnp.float32)],
    )

    dense2 = pl.pallas_call(
        _dense2_kernel,
        grid=(_N // _ROWB,),
        in_specs=[_row_spec(_NHID), _row_spec(_NHID),
                  _full_spec(_NHID, 3 * _NHID), _full_spec(_NHID, 3 * _NHID),
                  _full_spec(1, 3 * _NHID), _full_spec(1, 3 * _NHID),
                  _row_spec(1)],
        out_specs=_row_spec(_NHID),
        out_shape=jax.ShapeDtypeStruct((_N, _NHID), jnp.float32),
    )

    for t in range(_T):
        whh, es, ed = dense1(feat, hp, Wg, A_src, A_dst)
        ga = es[src] + ed[dst]
        e = _edge_ew(_leaky_kernel, None, ga)
        emax = jax.ops.segment_max(e, dst, num_segments=_N)
        emax = jnp.where(jnp.isfinite(emax), emax, 0.0)
        ee = _edge_ew(_ee_kernel, None, e, emax[dst])
        esum = jax.ops.segment_sum(ee, dst, num_segments=_N)
        att = _edge_ew(_att_kernel, None, ee, esum[dst])
        msg = att[:, :, None] * whh[src].reshape(_E, _H, _DH)
        agg = jax.ops.segment_sum(msg, dst, num_segments=_N).reshape(_N, _NHID)
        mask = (ids == t).astype(jnp.float32).reshape(_N, 1)
        hp = dense2(agg, hp, Wi, Uh, bi2, bh2, mask)

    s1, s2 = pl.pallas_call(
        _readout_kernel,
        grid=(1,),
        in_specs=[_full_spec(_N, _NHID), _full_spec(_N, 1),
                  _full_spec(_NHID, _NHID // 2), _full_spec(1, _NHID // 2),
                  _full_spec(_NHID // 2, 1), _full_spec(1, 1)],
        out_specs=[_full_spec(1, 1), _full_spec(1, 1)],
        out_shape=[jax.ShapeDtypeStruct((1, 1), jnp.float32),
                   jax.ShapeDtypeStruct((1, 1), jnp.float32)],
    )(hp, ids2, Wfc, bfc.reshape(1, _NHID // 2), Wsc, bsc.reshape(1, 1))

    return (s1.reshape(()), s2.reshape(()))
